# baseline (speedup 1.0000x reference)
import functools

import jax
import jax.numpy as jnp
from jax import lax
from jax.experimental import pallas as pl
from jax.experimental.pallas import tpu as pltpu
from jax.experimental.pallas import tpu_sc as plsc

VOCAB = 100000
D = 32
CTX = 20
B = 1024

NC = 2
NS = 16
NW = NC * NS
B_PER_W = B // NW
IDX_PER_W = B_PER_W * CTX
GCHUNK = 80
NCHUNK = IDX_PER_W // GCHUNK
B_PER_CHUNK = GCHUNK // CTX

VB = 2048
NBLK = (VOCAB + VB - 1) // VB


def _sc_gather_sum(idx2d, table):
    mesh = plsc.VectorSubcoreMesh(core_axis_name="c", subcore_axis_name="s")

    @functools.partial(
        pl.kernel,
        mesh=mesh,
        out_type=jax.ShapeDtypeStruct((B, D), jnp.float32),
        scratch_types=[
            pltpu.VMEM((NCHUNK, GCHUNK), jnp.int32),
            pltpu.VMEM((IDX_PER_W, D), jnp.float32),
            pltpu.VMEM((B_PER_W, D), jnp.float32),
            pltpu.SemaphoreType.DMA,
        ],
        compiler_params=pltpu.CompilerParams(use_tc_tiling_on_sc=False),
    )
    def sc_kernel(idx_hbm, table_hbm, out_hbm, idx_v, rows_v, s_v, sem):
        wid = lax.axis_index("s") * NC + lax.axis_index("c")
        pltpu.sync_copy(idx_hbm.at[pl.ds(wid * NCHUNK, NCHUNK)], idx_v)
        for j in range(NCHUNK):
            pltpu.async_copy(
                table_hbm.at[idx_v.at[j]],
                rows_v.at[pl.ds(j * GCHUNK, GCHUNK)],
                sem,
            )
        for j in range(NCHUNK):
            pltpu.make_async_copy(
                table_hbm.at[idx_v.at[j]],
                rows_v.at[pl.ds(j * GCHUNK, GCHUNK)],
                sem,
            ).wait()

        @pl.loop(0, B_PER_W)
        def _(g):
            base = g * CTX
            for h in range(D // 16):
                sl = pl.ds(h * 16, 16)
                acc = rows_v[base, sl]
                for c in range(1, CTX):
                    acc = acc + rows_v[base + c, sl]
                s_v[g, sl] = acc

        pltpu.sync_copy(s_v, out_hbm.at[pl.ds(wid * B_PER_W, B_PER_W)])

    return sc_kernel(idx2d, table)


def _pass1_body(s_ref, w_ref, b_ref, norm_ref, l_ref):
    i = pl.program_id(0)

    @pl.when(i == 0)
    def _():
        l_ref[...] = jnp.zeros_like(l_ref)

    sb = s_ref[...].astype(jnp.bfloat16)
    wb = w_ref[...].astype(jnp.bfloat16)
    logits = lax.dot_general(
        sb, wb, (((1,), (1,)), ((), ())), preferred_element_type=jnp.float32
    )
    e = jnp.exp(logits + b_ref[...])
    cols = lax.broadcasted_iota(jnp.int32, (1, VB), 1) + i * VB
    e = jnp.where(cols < VOCAB, e, 0.0)
    l_ref[...] += jnp.sum(e, axis=1, keepdims=True)

    @pl.when(i == NBLK - 1)
    def _():
        norm_ref[...] = jnp.log(l_ref[...])


def _pass2_body(s_ref, w_ref, b_ref, norm_ref, out_ref):
    sb = s_ref[...].astype(jnp.bfloat16)
    wb = w_ref[...].astype(jnp.bfloat16)
    logits = lax.dot_general(
        sb, wb, (((1,), (1,)), ((), ())), preferred_element_type=jnp.float32
    )
    out_ref[...] = logits + b_ref[...] - norm_ref[...]


def kernel(tokens, embed_table, W, b):
    idx2d = tokens.astype(jnp.int32).T.reshape(B * CTX // GCHUNK, GCHUNK)
    s = _sc_gather_sum(idx2d, embed_table)
    b2 = b.reshape(1, VOCAB)

    norm = pl.pallas_call(
        _pass1_body,
        grid=(NBLK,),
        in_specs=[
            pl.BlockSpec((B, D), lambda i: (0, 0)),
            pl.BlockSpec((VB, D), lambda i: (i, 0)),
            pl.BlockSpec((1, VB), lambda i: (0, i)),
        ],
        out_specs=pl.BlockSpec((B, 1), lambda i: (0, 0)),
        out_shape=jax.ShapeDtypeStruct((B, 1), jnp.float32),
        scratch_shapes=[pltpu.VMEM((B, 1), jnp.float32)],
        compiler_params=pltpu.CompilerParams(
            dimension_semantics=("arbitrary",)
        ),
    )(s, W, b2)

    out = pl.pallas_call(
        _pass2_body,
        grid=(NBLK,),
        in_specs=[
            pl.BlockSpec((B, D), lambda i: (0, 0)),
            pl.BlockSpec((VB, D), lambda i: (i, 0)),
            pl.BlockSpec((1, VB), lambda i: (0, i)),
            pl.BlockSpec((B, 1), lambda i: (0, 0)),
        ],
        out_specs=pl.BlockSpec((B, VB), lambda i: (0, i)),
        out_shape=jax.ShapeDtypeStruct((B, VOCAB), jnp.float32),
        compiler_params=pltpu.CompilerParams(
            dimension_semantics=("arbitrary",)
        ),
    )(s, W, b2, norm)
    return out


# speedup vs baseline: 1.9449x; 1.9449x over previous
import functools

import jax
import jax.numpy as jnp
from jax import lax
from jax.experimental import pallas as pl
from jax.experimental.pallas import tpu as pltpu
from jax.experimental.pallas import tpu_sc as plsc

VOCAB = 100000
D = 32
CTX = 20
B = 1024

NC = 2
NS = 16
NW = NC * NS
B_PER_W = B // NW
IDX_PER_W = B_PER_W * CTX
GCHUNK = 80
NCHUNK = IDX_PER_W // GCHUNK

VB = 2048
NBLK = (VOCAB + VB - 1) // VB


def _sc_gather_sum(idx2d, table):
    mesh = plsc.VectorSubcoreMesh(core_axis_name="c", subcore_axis_name="s")

    @functools.partial(
        pl.kernel,
        mesh=mesh,
        out_type=jax.ShapeDtypeStruct((B, D), jnp.float32),
        scratch_types=[
            pltpu.VMEM((NCHUNK, GCHUNK), jnp.int32),
            pltpu.VMEM((IDX_PER_W, D), jnp.float32),
            pltpu.VMEM((B_PER_W, D), jnp.float32),
            pltpu.SemaphoreType.DMA,
        ],
        compiler_params=pltpu.CompilerParams(use_tc_tiling_on_sc=False),
    )
    def sc_kernel(idx_hbm, table_hbm, out_hbm, idx_v, rows_v, s_v, sem):
        wid = lax.axis_index("s") * NC + lax.axis_index("c")
        pltpu.sync_copy(idx_hbm.at[pl.ds(wid * NCHUNK, NCHUNK)], idx_v)
        for j in range(NCHUNK):
            pltpu.async_copy(
                table_hbm.at[idx_v.at[j]],
                rows_v.at[pl.ds(j * GCHUNK, GCHUNK)],
                sem,
            )
        for j in range(NCHUNK):
            pltpu.make_async_copy(
                table_hbm.at[idx_v.at[j]],
                rows_v.at[pl.ds(j * GCHUNK, GCHUNK)],
                sem,
            ).wait()

        @pl.loop(0, B_PER_W)
        def _(g):
            base = g * CTX
            for h in range(D // 16):
                sl = pl.ds(h * 16, 16)
                acc = rows_v[base, sl]
                for c in range(1, CTX):
                    acc = acc + rows_v[base + c, sl]
                s_v[g, sl] = acc

        pltpu.sync_copy(s_v, out_hbm.at[pl.ds(wid * B_PER_W, B_PER_W)])

    return sc_kernel(idx2d, table)


def _logits_t(wt_ref, s_ref, b_ref):
    wb = wt_ref[...].astype(jnp.bfloat16)
    sb = s_ref[...].astype(jnp.bfloat16)
    lt = lax.dot_general(
        wb, sb, (((0,), (1,)), ((), ())), preferred_element_type=jnp.float32
    )
    return lt + b_ref[...]


def _pass1_body(wt_ref, s_ref, b_ref, norm_ref, l_ref):
    i = pl.program_id(0)

    @pl.when(i == 0)
    def _():
        l_ref[...] = jnp.zeros_like(l_ref)

    e = jnp.exp(_logits_t(wt_ref, s_ref, b_ref))
    rows = lax.broadcasted_iota(jnp.int32, (VB, 1), 0) + i * VB
    e = jnp.where(rows < VOCAB, e, 0.0)
    l_ref[...] += jnp.sum(e, axis=0, keepdims=True)

    @pl.when(i == NBLK - 1)
    def _():
        norm_ref[...] = jnp.log(l_ref[...])


def _pass2_body(wt_ref, s_ref, b_ref, norm_ref, out_ref):
    out_ref[...] = _logits_t(wt_ref, s_ref, b_ref) - norm_ref[...]


def kernel(tokens, embed_table, W, b):
    idx2d = tokens.astype(jnp.int32).T.reshape(B * CTX // GCHUNK, GCHUNK)
    s = _sc_gather_sum(idx2d, embed_table)
    wt = W.T
    bc = b.reshape(VOCAB, 1)

    norm = pl.pallas_call(
        _pass1_body,
        grid=(NBLK,),
        in_specs=[
            pl.BlockSpec((D, VB), lambda i: (0, i)),
            pl.BlockSpec((B, D), lambda i: (0, 0)),
            pl.BlockSpec((VB, 1), lambda i: (i, 0)),
        ],
        out_specs=pl.BlockSpec((1, B), lambda i: (0, 0)),
        out_shape=jax.ShapeDtypeStruct((1, B), jnp.float32),
        scratch_shapes=[pltpu.VMEM((1, B), jnp.float32)],
        compiler_params=pltpu.CompilerParams(
            dimension_semantics=("arbitrary",)
        ),
    )(wt, s, bc)

    out_t = pl.pallas_call(
        _pass2_body,
        grid=(NBLK,),
        in_specs=[
            pl.BlockSpec((D, VB), lambda i: (0, i)),
            pl.BlockSpec((B, D), lambda i: (0, 0)),
            pl.BlockSpec((VB, 1), lambda i: (i, 0)),
            pl.BlockSpec((1, B), lambda i: (0, 0)),
        ],
        out_specs=pl.BlockSpec((VB, B), lambda i: (i, 0)),
        out_shape=jax.ShapeDtypeStruct((VOCAB, B), jnp.float32),
        compiler_params=pltpu.CompilerParams(
            dimension_semantics=("arbitrary",)
        ),
    )(wt, s, bc, norm)
    return out_t.T
